# baseline (speedup 1.0000x reference)
import functools
import math

import jax
import jax.numpy as jnp
from jax import lax
from jax.experimental import pallas as pl
from jax.experimental.pallas import tpu as pltpu
from jax.experimental.pallas import tpu_sc as plsc

_NC = 2
_NS = 16
_NW = _NC * _NS
_CHUNK = 128

_PACK_NC = 16384
_PACK_H = _PACK_NC // 2


def _pack_rows(V):
    tail = V - (V // _PACK_NC) * _PACK_NC
    return (V // _PACK_NC) * _PACK_H + tail


_HI_MASK = -65536


def _pack_bf16(block):
    h = block.shape[1] // 2
    rb = lambda x: lax.bitcast_convert_type(x, jnp.int32) + 32768
    hi = lax.bitwise_and(rb(block[:, :h]), _HI_MASK)
    lo = lax.shift_right_logical(rb(block[:, h:]), 16)
    return lax.bitcast_convert_type(lax.bitwise_or(hi, lo), jnp.float32)


def _unpack_bf16(pk):
    pki = lax.bitcast_convert_type(pk, jnp.int32)
    hi = lax.bitcast_convert_type(lax.bitwise_and(pki, _HI_MASK), jnp.float32)
    lo = lax.bitcast_convert_type(lax.shift_left(pki, 16), jnp.float32)
    return hi, lo


def _pack_body(wt_ref, out_ref):
    Dh = wt_ref.shape[0] // 2
    xt = wt_ref[...].T
    out_ref[:, :Dh] = _pack_bf16(xt[:_PACK_H])
    out_ref[:, Dh:] = _pack_bf16(xt[_PACK_H:])


def _tc_pack(weightT):
    D, V = weightT.shape
    return pl.pallas_call(
        _pack_body,
        grid=(pl.cdiv(V, _PACK_NC),),
        in_specs=[pl.BlockSpec((D, _PACK_NC), lambda j: (0, j))],
        out_specs=pl.BlockSpec((_PACK_H, D), lambda j: (j, 0)),
        out_shape=jax.ShapeDtypeStruct((_pack_rows(V), D), jnp.float32),
        compiler_params=pltpu.CompilerParams(
            dimension_semantics=("arbitrary",),
        ),
    )(weightT)


def _sc_gather(w2, bias, nlp, target, samples, log_ns):
    D2 = w2.shape[1]
    B = target.shape[0]
    S = samples.shape[0]
    sh_blk = _PACK_NC.bit_length() - 1
    sh_half = _PACK_H.bit_length() - 1
    mesh = plsc.VectorSubcoreMesh(core_axis_name="c", subcore_axis_name="s")

    @functools.partial(
        pl.kernel,
        mesh=mesh,
        out_type=[
            jax.ShapeDtypeStruct((B, D2), jnp.float32),
            jax.ShapeDtypeStruct((S, D2), jnp.float32),
            jax.ShapeDtypeStruct((B,), jnp.float32),
            jax.ShapeDtypeStruct((S,), jnp.float32),
            jax.ShapeDtypeStruct((B,), jnp.float32),
            jax.ShapeDtypeStruct((S,), jnp.float32),
        ],
        scratch_types=[
            pltpu.VMEM((_CHUNK,), jnp.int32),
            pltpu.VMEM((_CHUNK,), jnp.int32),
            pltpu.VMEM((_CHUNK, D2), jnp.float32),
            pltpu.VMEM((_CHUNK,), jnp.float32),
            pltpu.VMEM((_CHUNK,), jnp.float32),
            pltpu.VMEM((_CHUNK,), jnp.float32),
            pltpu.VMEM((_CHUNK,), jnp.float32),
            pltpu.SemaphoreType.DMA,
        ],
        compiler_params=pltpu.CompilerParams(use_tc_tiling_on_sc=False),
    )
    def gather_kernel(w2_hbm, bias_hbm, nlp_hbm, target_hbm, samples_hbm,
                      trows_hbm, srows_hbm, toff_hbm, soff_hbm,
                      tpar_hbm, spar_hbm,
                      idx_v, slot_v, rows_v, b_v, n_v, o_v, p_v, sem):
        wid = lax.axis_index("s") * _NC + lax.axis_index("c")

        def do_chunk(idx_hbm, rows_out, off_out, par_out, base):
            pltpu.sync_copy(idx_hbm.at[pl.ds(base, _CHUNK)], idx_v)
            for k in range(_CHUNK // 16):
                sl = pl.ds(k * 16, 16)
                c = idx_v[sl]
                slot_v[sl] = lax.bitwise_or(
                    lax.shift_left(lax.shift_right_logical(c, sh_blk),
                                   sh_half),
                    lax.bitwise_and(c, _PACK_H - 1))
                p_v[sl] = lax.convert_element_type(
                    lax.bitwise_and(lax.shift_right_logical(c, sh_half), 1),
                    jnp.float32)
            pltpu.async_copy(w2_hbm.at[slot_v], rows_v, sem).wait()
            pltpu.async_copy(bias_hbm.at[idx_v], b_v, sem).wait()
            pltpu.async_copy(nlp_hbm.at[idx_v], n_v, sem).wait()
            for k in range(_CHUNK // 16):
                sl = pl.ds(k * 16, 16)
                o_v[sl] = b_v[sl] - (n_v[sl] + log_ns)
            pltpu.sync_copy(rows_v, rows_out.at[pl.ds(base, _CHUNK)])
            pltpu.sync_copy(o_v, off_out.at[pl.ds(base, _CHUNK)])
            pltpu.sync_copy(p_v, par_out.at[pl.ds(base, _CHUNK)])

        for c in range(B // (_NW * _CHUNK)):
            do_chunk(target_hbm, trows_hbm, toff_hbm, tpar_hbm,
                     wid * (B // _NW) + c * _CHUNK)
        for c in range(S // (_NW * _CHUNK)):
            do_chunk(samples_hbm, srows_hbm, soff_hbm, spar_hbm,
                     wid * (S // _NW) + c * _CHUNK)

    return gather_kernel(w2, bias, nlp, target, samples)


_LOG2E = 1.4426950408889634
_LN2 = 0.6931471805599453


def _softplus_neg_abs(x):
    p = jnp.exp2(jnp.abs(x) * (-_LOG2E))
    return _LN2 * jnp.log2(1.0 + p)


def _tc_loss_T_body(se_ref, dt_ref, tt_ref, to_ref, tp_ref, so_ref, sp_ref,
                    out_ref):
    j = pl.program_id(0)
    D = dt_ref.shape[0]
    Dh = D // 2
    dt = dt_ref[...]
    se = se_ref[...]
    pk = jnp.where(sp_ref[...] < 0.5, se[:, :Dh], se[:, Dh:])
    ehi, elo = _unpack_bf16(pk)
    dn = (((1,), (0,)), ((), ()))
    mm = (lax.dot_general(ehi, dt[:Dh], dimension_numbers=dn,
                          preferred_element_type=jnp.float32)
          + lax.dot_general(elo, dt[Dh:], dimension_numbers=dn,
                            preferred_element_type=jnp.float32))
    sl = mm + so_ref[...]
    out_ref[...] = jnp.maximum(sl, 0.0) + _softplus_neg_abs(sl)

    @pl.when(j == 0)
    def _():
        tt = tt_ref[...]
        tpk = jnp.where(tp_ref[...] < 0.5, tt[:Dh, :], tt[Dh:, :])
        thi, tlo = _unpack_bf16(tpk)
        tl = jnp.sum(dt[:Dh] * thi + dt[Dh:] * tlo, axis=0, keepdims=True)
        tl = tl + to_ref[...]
        out_ref[0:1, :] = jnp.maximum(tl, 0.0) - tl + _softplus_neg_abs(tl)


def _tc_loss_T(dataT, srows_p, soff_p, spar_p, trowsT, toff_row, tpar_row,
               bt=512):
    D, B = dataT.shape
    D2 = srows_p.shape[1]
    S1 = srows_p.shape[0]
    return pl.pallas_call(
        _tc_loss_T_body,
        grid=(pl.cdiv(S1, bt),),
        in_specs=[
            pl.BlockSpec((bt, D2), lambda j: (j, 0)),
            pl.BlockSpec((D, B), lambda j: (0, 0)),
            pl.BlockSpec((D2, B), lambda j: (0, 0)),
            pl.BlockSpec((1, B), lambda j: (0, 0)),
            pl.BlockSpec((1, B), lambda j: (0, 0)),
            pl.BlockSpec((bt, 1), lambda j: (j, 0)),
            pl.BlockSpec((bt, 1), lambda j: (j, 0)),
        ],
        out_specs=pl.BlockSpec((bt, B), lambda j: (j, 0)),
        out_shape=jax.ShapeDtypeStruct((S1, B), jnp.float32),
        compiler_params=pltpu.CompilerParams(
            dimension_semantics=("arbitrary",),
        ),
    )(srows_p, dataT, trowsT, toff_row, tpar_row, soff_p, spar_p)


def kernel(data, target, samples, weight, bias, noise_log_probs):
    B = data.shape[0]
    S = samples.shape[0]
    log_ns = math.log(S)
    w2 = _tc_pack(weight.T)
    trows, srows, toff, soff, tpar, spar = _sc_gather(
        w2, bias, noise_log_probs,
        target.astype(jnp.int32), samples.astype(jnp.int32), log_ns)
    srows_p = jnp.pad(srows, ((1, 0), (0, 0)))
    soff_p = jnp.pad(soff, (1, 0)).reshape(S + 1, 1)
    spar_p = jnp.pad(spar, (1, 0)).reshape(S + 1, 1)
    out_T = _tc_loss_T(data.T, srows_p, soff_p, spar_p, trows.T,
                       toff.reshape(1, B), tpar.reshape(1, B))
    return out_T.T


# speedup vs baseline: 1.3132x; 1.3132x over previous
import functools
import math

import jax
import jax.numpy as jnp
from jax import lax
from jax.experimental import pallas as pl
from jax.experimental.pallas import tpu as pltpu
from jax.experimental.pallas import tpu_sc as plsc

_NC = 2
_NS = 16
_NW = _NC * _NS
_CHUNK = 128

_PACK_NC = 16384
_PACK_Q = _PACK_NC // 4


def _pack_rows(V):
    tail = V - (V // _PACK_NC) * _PACK_NC
    return (V // _PACK_NC) * _PACK_Q + tail


_HI_MASK = -65536


def _pack_bf16(block):
    h = block.shape[1] // 2
    rb = lambda x: lax.bitcast_convert_type(x, jnp.int32) + 32768
    hi = lax.bitwise_and(rb(block[:, :h]), _HI_MASK)
    lo = lax.shift_right_logical(rb(block[:, h:]), 16)
    return lax.bitcast_convert_type(lax.bitwise_or(hi, lo), jnp.float32)


def _unpack_bf16(pk):
    pki = lax.bitcast_convert_type(pk, jnp.int32)
    hi = lax.bitcast_convert_type(lax.bitwise_and(pki, _HI_MASK), jnp.float32)
    lo = lax.bitcast_convert_type(lax.shift_left(pki, 16), jnp.float32)
    return hi, lo


def _pack_body(wt_ref, out_ref):
    Dh = wt_ref.shape[0] // 2
    xt = wt_ref[...].T
    for q in range(4):
        out_ref[:, q * Dh:(q + 1) * Dh] = _pack_bf16(
            xt[q * _PACK_Q:(q + 1) * _PACK_Q])


def _tc_pack(weightT):
    D, V = weightT.shape
    return pl.pallas_call(
        _pack_body,
        grid=(pl.cdiv(V, _PACK_NC),),
        in_specs=[pl.BlockSpec((D, _PACK_NC), lambda j: (0, j))],
        out_specs=pl.BlockSpec((_PACK_Q, 2 * D), lambda j: (j, 0)),
        out_shape=jax.ShapeDtypeStruct((_pack_rows(V), 2 * D), jnp.float32),
        compiler_params=pltpu.CompilerParams(
            dimension_semantics=("arbitrary",),
        ),
    )(weightT)


def _sc_gather(w2, bias, nlp, target, samples, log_ns):
    D2 = w2.shape[1]
    B = target.shape[0]
    S = samples.shape[0]
    sh_blk = _PACK_NC.bit_length() - 1
    sh_q = _PACK_Q.bit_length() - 1
    mesh = plsc.VectorSubcoreMesh(core_axis_name="c", subcore_axis_name="s")

    @functools.partial(
        pl.kernel,
        mesh=mesh,
        out_type=[
            jax.ShapeDtypeStruct((B, D2), jnp.float32),
            jax.ShapeDtypeStruct((S, D2), jnp.float32),
            jax.ShapeDtypeStruct((B,), jnp.float32),
            jax.ShapeDtypeStruct((S,), jnp.float32),
            jax.ShapeDtypeStruct((B,), jnp.float32),
            jax.ShapeDtypeStruct((S,), jnp.float32),
        ],
        scratch_types=[
            pltpu.VMEM((_CHUNK,), jnp.int32),
            pltpu.VMEM((_CHUNK,), jnp.int32),
            pltpu.VMEM((_CHUNK, D2), jnp.float32),
            pltpu.VMEM((_CHUNK,), jnp.float32),
            pltpu.VMEM((_CHUNK,), jnp.float32),
            pltpu.VMEM((_CHUNK,), jnp.float32),
            pltpu.VMEM((_CHUNK,), jnp.float32),
            pltpu.SemaphoreType.DMA,
        ],
        compiler_params=pltpu.CompilerParams(use_tc_tiling_on_sc=False),
    )
    def gather_kernel(w2_hbm, bias_hbm, nlp_hbm, target_hbm, samples_hbm,
                      trows_hbm, srows_hbm, toff_hbm, soff_hbm,
                      tpar_hbm, spar_hbm,
                      idx_v, slot_v, rows_v, b_v, n_v, o_v, p_v, sem):
        wid = lax.axis_index("s") * _NC + lax.axis_index("c")

        def do_chunk(idx_hbm, rows_out, off_out, par_out, base):
            pltpu.sync_copy(idx_hbm.at[pl.ds(base, _CHUNK)], idx_v)
            for k in range(_CHUNK // 16):
                sl = pl.ds(k * 16, 16)
                c = idx_v[sl]
                slot_v[sl] = lax.bitwise_or(
                    lax.shift_left(lax.shift_right_logical(c, sh_blk),
                                   sh_q),
                    lax.bitwise_and(c, _PACK_Q - 1))
                p_v[sl] = lax.convert_element_type(
                    lax.bitwise_and(lax.shift_right_logical(c, sh_q), 3),
                    jnp.float32)
            pltpu.async_copy(w2_hbm.at[slot_v], rows_v, sem).wait()
            pltpu.async_copy(bias_hbm.at[idx_v], b_v, sem).wait()
            pltpu.async_copy(nlp_hbm.at[idx_v], n_v, sem).wait()
            for k in range(_CHUNK // 16):
                sl = pl.ds(k * 16, 16)
                o_v[sl] = b_v[sl] - (n_v[sl] + log_ns)
            pltpu.sync_copy(rows_v, rows_out.at[pl.ds(base, _CHUNK)])
            pltpu.sync_copy(o_v, off_out.at[pl.ds(base, _CHUNK)])
            pltpu.sync_copy(p_v, par_out.at[pl.ds(base, _CHUNK)])

        for c in range(B // (_NW * _CHUNK)):
            do_chunk(target_hbm, trows_hbm, toff_hbm, tpar_hbm,
                     wid * (B // _NW) + c * _CHUNK)
        for c in range(S // (_NW * _CHUNK)):
            do_chunk(samples_hbm, srows_hbm, soff_hbm, spar_hbm,
                     wid * (S // _NW) + c * _CHUNK)

    return gather_kernel(w2, bias, nlp, target, samples)


_LOG2E = 1.4426950408889634
_LN2 = 0.6931471805599453


def _softplus_neg_abs(x):
    p = jnp.exp2(jnp.abs(x) * (-_LOG2E))
    return _LN2 * jnp.log2(1.0 + p)


def _tc_loss_T_body(se_ref, dt_ref, tt_ref, to_ref, tp_ref, so_ref, sp_ref,
                    out_ref):
    j = pl.program_id(0)
    D = dt_ref.shape[0]
    Dh = D // 2
    dt = dt_ref[...]
    se = se_ref[...]
    q = sp_ref[...]
    pk = jnp.where(
        q < 1.5,
        jnp.where(q < 0.5, se[:, :Dh], se[:, Dh:2 * Dh]),
        jnp.where(q < 2.5, se[:, 2 * Dh:3 * Dh], se[:, 3 * Dh:]))
    ehi, elo = _unpack_bf16(pk)
    dn = (((1,), (0,)), ((), ()))
    mm = (lax.dot_general(ehi, dt[:Dh], dimension_numbers=dn,
                          preferred_element_type=jnp.float32)
          + lax.dot_general(elo, dt[Dh:], dimension_numbers=dn,
                            preferred_element_type=jnp.float32))
    sl = mm + so_ref[...]
    out_ref[...] = jnp.maximum(sl, 0.0) + _softplus_neg_abs(sl)

    @pl.when(j == 0)
    def _():
        tt = tt_ref[...]
        tq = tp_ref[...]
        tpk = jnp.where(
            tq < 1.5,
            jnp.where(tq < 0.5, tt[:Dh, :], tt[Dh:2 * Dh, :]),
            jnp.where(tq < 2.5, tt[2 * Dh:3 * Dh, :], tt[3 * Dh:, :]))
        thi, tlo = _unpack_bf16(tpk)
        tl = jnp.sum(dt[:Dh] * thi + dt[Dh:] * tlo, axis=0, keepdims=True)
        tl = tl + to_ref[...]
        out_ref[0:1, :] = jnp.maximum(tl, 0.0) - tl + _softplus_neg_abs(tl)


def _tc_loss_T(dataT, srows_p, soff_p, spar_p, trowsT, toff_row, tpar_row,
               bt=512):
    D, B = dataT.shape
    D2 = srows_p.shape[1]
    S1 = srows_p.shape[0]
    return pl.pallas_call(
        _tc_loss_T_body,
        grid=(pl.cdiv(S1, bt),),
        in_specs=[
            pl.BlockSpec((bt, D2), lambda j: (j, 0)),
            pl.BlockSpec((D, B), lambda j: (0, 0)),
            pl.BlockSpec((D2, B), lambda j: (0, 0)),
            pl.BlockSpec((1, B), lambda j: (0, 0)),
            pl.BlockSpec((1, B), lambda j: (0, 0)),
            pl.BlockSpec((bt, 1), lambda j: (j, 0)),
            pl.BlockSpec((bt, 1), lambda j: (j, 0)),
        ],
        out_specs=pl.BlockSpec((bt, B), lambda j: (j, 0)),
        out_shape=jax.ShapeDtypeStruct((S1, B), jnp.float32),
        compiler_params=pltpu.CompilerParams(
            dimension_semantics=("arbitrary",),
        ),
    )(srows_p, dataT, trowsT, toff_row, tpar_row, soff_p, spar_p)


def kernel(data, target, samples, weight, bias, noise_log_probs):
    B = data.shape[0]
    S = samples.shape[0]
    log_ns = math.log(S)
    w2 = _tc_pack(weight.T)
    trows, srows, toff, soff, tpar, spar = _sc_gather(
        w2, bias, noise_log_probs,
        target.astype(jnp.int32), samples.astype(jnp.int32), log_ns)
    srows_p = jnp.pad(srows, ((1, 0), (0, 0)))
    soff_p = jnp.pad(soff, (1, 0)).reshape(S + 1, 1)
    spar_p = jnp.pad(spar, (1, 0)).reshape(S + 1, 1)
    out_T = _tc_loss_T(data.T, srows_p, soff_p, spar_p, trows.T,
                       toff.reshape(1, B), tpar.reshape(1, B))
    return out_T.T


# speedup vs baseline: 2.2069x; 1.6806x over previous
import functools
import math

import jax
import jax.numpy as jnp
from jax import lax
from jax.experimental import pallas as pl
from jax.experimental.pallas import tpu as pltpu
from jax.experimental.pallas import tpu_sc as plsc

_NC = 2
_NS = 16
_NW = _NC * _NS
_CHUNK = 128

_PACK_NC = 16384
_PACK_Q = _PACK_NC // 4


def _pack_rows(V):
    tail = V - (V // _PACK_NC) * _PACK_NC
    return (V // _PACK_NC) * _PACK_Q + tail


_HI_MASK = -65536


def _pack_bf16(xhi, xlo):
    rb = lambda x: lax.bitcast_convert_type(x, jnp.int32) + 32768
    hi = lax.bitwise_and(rb(xhi), _HI_MASK)
    lo = lax.shift_right_logical(rb(xlo), 16)
    return lax.bitcast_convert_type(lax.bitwise_or(hi, lo), jnp.float32)


def _unpack_bf16(pk):
    pki = lax.bitcast_convert_type(pk, jnp.int32)
    hi = lax.bitcast_convert_type(lax.bitwise_and(pki, _HI_MASK), jnp.float32)
    lo = lax.bitcast_convert_type(lax.shift_left(pki, 16), jnp.float32)
    return hi, lo


def _pack_body(wt_ref, out_ref):
    Q = _PACK_Q
    xt = wt_ref[...].T
    a = _pack_bf16(xt[:Q], xt[Q:2 * Q])
    b = _pack_bf16(xt[2 * Q:3 * Q], xt[3 * Q:])
    out_ref[...] = jnp.concatenate([a, b], axis=1)


def _tc_pack(weightT):
    D, V = weightT.shape
    return pl.pallas_call(
        _pack_body,
        grid=(pl.cdiv(V, _PACK_NC),),
        in_specs=[pl.BlockSpec((D, _PACK_NC), lambda j: (0, j))],
        out_specs=pl.BlockSpec((_PACK_Q, 2 * D), lambda j: (j, 0)),
        out_shape=jax.ShapeDtypeStruct((_pack_rows(V), 2 * D), jnp.float32),
        compiler_params=pltpu.CompilerParams(
            dimension_semantics=("arbitrary",),
        ),
    )(weightT)


def _sc_gather(w2, bias, nlp, target, samples, log_ns):
    D2 = w2.shape[1]
    B = target.shape[0]
    S = samples.shape[0]
    sh_blk = _PACK_NC.bit_length() - 1
    sh_q = _PACK_Q.bit_length() - 1
    mesh = plsc.VectorSubcoreMesh(core_axis_name="c", subcore_axis_name="s")

    @functools.partial(
        pl.kernel,
        mesh=mesh,
        out_type=[
            jax.ShapeDtypeStruct((B, D2), jnp.float32),
            jax.ShapeDtypeStruct((S, D2), jnp.float32),
            jax.ShapeDtypeStruct((B,), jnp.float32),
            jax.ShapeDtypeStruct((S,), jnp.float32),
            jax.ShapeDtypeStruct((B,), jnp.float32),
            jax.ShapeDtypeStruct((S,), jnp.float32),
        ],
        scratch_types=[
            pltpu.VMEM((_CHUNK,), jnp.int32),
            pltpu.VMEM((_CHUNK,), jnp.int32),
            pltpu.VMEM((_CHUNK, D2), jnp.float32),
            pltpu.VMEM((_CHUNK,), jnp.float32),
            pltpu.VMEM((_CHUNK,), jnp.float32),
            pltpu.VMEM((_CHUNK,), jnp.float32),
            pltpu.VMEM((_CHUNK,), jnp.float32),
            pltpu.SemaphoreType.DMA,
        ],
        compiler_params=pltpu.CompilerParams(use_tc_tiling_on_sc=False),
    )
    def gather_kernel(w2_hbm, bias_hbm, nlp_hbm, target_hbm, samples_hbm,
                      trows_hbm, srows_hbm, toff_hbm, soff_hbm,
                      tpar_hbm, spar_hbm,
                      idx_v, slot_v, rows_v, b_v, n_v, o_v, p_v, sem):
        wid = lax.axis_index("s") * _NC + lax.axis_index("c")

        def do_chunk(idx_hbm, rows_out, off_out, par_out, base):
            pltpu.sync_copy(idx_hbm.at[pl.ds(base, _CHUNK)], idx_v)
            for k in range(_CHUNK // 16):
                sl = pl.ds(k * 16, 16)
                c = idx_v[sl]
                slot_v[sl] = lax.bitwise_or(
                    lax.shift_left(lax.shift_right_logical(c, sh_blk),
                                   sh_q),
                    lax.bitwise_and(c, _PACK_Q - 1))
                p_v[sl] = lax.convert_element_type(
                    lax.bitwise_and(lax.shift_right_logical(c, sh_q), 3),
                    jnp.float32)
            pltpu.async_copy(w2_hbm.at[slot_v], rows_v, sem).wait()
            pltpu.async_copy(bias_hbm.at[idx_v], b_v, sem).wait()
            pltpu.async_copy(nlp_hbm.at[idx_v], n_v, sem).wait()
            for k in range(_CHUNK // 16):
                sl = pl.ds(k * 16, 16)
                o_v[sl] = b_v[sl] - (n_v[sl] + log_ns)
            pltpu.sync_copy(rows_v, rows_out.at[pl.ds(base, _CHUNK)])
            pltpu.sync_copy(o_v, off_out.at[pl.ds(base, _CHUNK)])
            pltpu.sync_copy(p_v, par_out.at[pl.ds(base, _CHUNK)])

        for c in range(B // (_NW * _CHUNK)):
            do_chunk(target_hbm, trows_hbm, toff_hbm, tpar_hbm,
                     wid * (B // _NW) + c * _CHUNK)
        for c in range(S // (_NW * _CHUNK)):
            do_chunk(samples_hbm, srows_hbm, soff_hbm, spar_hbm,
                     wid * (S // _NW) + c * _CHUNK)

    return gather_kernel(w2, bias, nlp, target, samples)


_LOG2E = 1.4426950408889634
_LN2 = 0.6931471805599453


def _softplus_neg_abs(x):
    p = jnp.exp2(jnp.abs(x) * (-_LOG2E))
    return _LN2 * jnp.log2(1.0 + p)


def _tc_loss_T_body(se_ref, dt_ref, tt_ref, to_ref, tp_ref, so_ref, sp_ref,
                    out_ref):
    j = pl.program_id(0)
    D = dt_ref.shape[0]
    Dh = D // 2
    dt = dt_ref[...]
    se = se_ref[...]
    q = sp_ref[...]
    hi_l, lo_l = _unpack_bf16(se[:, :D])
    hi_u, lo_u = _unpack_bf16(se[:, D:])
    emb = jnp.where(q < 1.5,
                    jnp.where(q < 0.5, hi_l, lo_l),
                    jnp.where(q < 2.5, hi_u, lo_u))
    mm = lax.dot_general(emb, dt,
                         dimension_numbers=(((1,), (0,)), ((), ())),
                         preferred_element_type=jnp.float32)
    sl = mm + so_ref[...]
    out_ref[...] = jnp.maximum(sl, 0.0) + _softplus_neg_abs(sl)

    @pl.when(j == 0)
    def _():
        tt = tt_ref[...]
        tq = tp_ref[...]
        thi_l, tlo_l = _unpack_bf16(tt[:D, :])
        thi_u, tlo_u = _unpack_bf16(tt[D:, :])
        temb = jnp.where(tq < 1.5,
                         jnp.where(tq < 0.5, thi_l, tlo_l),
                         jnp.where(tq < 2.5, thi_u, tlo_u))
        tl = jnp.sum(dt * temb, axis=0, keepdims=True)
        tl = tl + to_ref[...]
        out_ref[0:1, :] = jnp.maximum(tl, 0.0) - tl + _softplus_neg_abs(tl)


def _tc_loss_T(dataT, srows_p, soff_p, spar_p, trowsT, toff_row, tpar_row,
               bt=512):
    D, B = dataT.shape
    D2 = srows_p.shape[1]
    S1 = srows_p.shape[0]
    return pl.pallas_call(
        _tc_loss_T_body,
        grid=(pl.cdiv(S1, bt),),
        in_specs=[
            pl.BlockSpec((bt, D2), lambda j: (j, 0)),
            pl.BlockSpec((D, B), lambda j: (0, 0)),
            pl.BlockSpec((D2, B), lambda j: (0, 0)),
            pl.BlockSpec((1, B), lambda j: (0, 0)),
            pl.BlockSpec((1, B), lambda j: (0, 0)),
            pl.BlockSpec((bt, 1), lambda j: (j, 0)),
            pl.BlockSpec((bt, 1), lambda j: (j, 0)),
        ],
        out_specs=pl.BlockSpec((bt, B), lambda j: (j, 0)),
        out_shape=jax.ShapeDtypeStruct((S1, B), jnp.float32),
        compiler_params=pltpu.CompilerParams(
            dimension_semantics=("arbitrary",),
        ),
    )(srows_p, dataT, trowsT, toff_row, tpar_row, soff_p, spar_p)


def kernel(data, target, samples, weight, bias, noise_log_probs):
    B = data.shape[0]
    S = samples.shape[0]
    log_ns = math.log(S)
    w2 = _tc_pack(weight.T)
    trows, srows, toff, soff, tpar, spar = _sc_gather(
        w2, bias, noise_log_probs,
        target.astype(jnp.int32), samples.astype(jnp.int32), log_ns)
    srows_p = jnp.pad(srows, ((1, 0), (0, 0)))
    soff_p = jnp.pad(soff, (1, 0)).reshape(S + 1, 1)
    spar_p = jnp.pad(spar, (1, 0)).reshape(S + 1, 1)
    out_T = _tc_loss_T(data.T, srows_p, soff_p, spar_p, trows.T,
                       toff.reshape(1, B), tpar.reshape(1, B))
    return out_T.T


# speedup vs baseline: 2.2489x; 1.0190x over previous
import functools
import math

import jax
import jax.numpy as jnp
from jax import lax
from jax.experimental import pallas as pl
from jax.experimental.pallas import tpu as pltpu
from jax.experimental.pallas import tpu_sc as plsc

_NC = 2
_NS = 16
_NW = _NC * _NS
_CHUNK = 128

_PACK_NC = 16384
_PACK_Q = _PACK_NC // 4


def _pack_rows(V):
    tail = V - (V // _PACK_NC) * _PACK_NC
    return (V // _PACK_NC) * _PACK_Q + tail


_HI_MASK = -65536


def _pack_bf16(xhi, xlo):
    rb = lambda x: lax.bitcast_convert_type(x, jnp.int32) + 32768
    hi = lax.bitwise_and(rb(xhi), _HI_MASK)
    lo = lax.shift_right_logical(rb(xlo), 16)
    return lax.bitcast_convert_type(lax.bitwise_or(hi, lo), jnp.float32)


def _unpack_bf16(pk):
    pki = lax.bitcast_convert_type(pk, jnp.int32)
    hi = lax.bitcast_convert_type(lax.bitwise_and(pki, _HI_MASK), jnp.float32)
    lo = lax.bitcast_convert_type(lax.shift_left(pki, 16), jnp.float32)
    return hi, lo


def _pack_body(wt_ref, out_ref):
    Q = _PACK_Q
    xt = wt_ref[...].T
    a = _pack_bf16(xt[:Q], xt[Q:2 * Q])
    b = _pack_bf16(xt[2 * Q:3 * Q], xt[3 * Q:])
    out_ref[...] = jnp.concatenate([a, b], axis=1)


def _tc_pack(weightT):
    D, V = weightT.shape
    return pl.pallas_call(
        _pack_body,
        grid=(pl.cdiv(V, _PACK_NC),),
        in_specs=[pl.BlockSpec((D, _PACK_NC), lambda j: (0, j))],
        out_specs=pl.BlockSpec((_PACK_Q, 2 * D), lambda j: (j, 0)),
        out_shape=jax.ShapeDtypeStruct((_pack_rows(V), 2 * D), jnp.float32),
        compiler_params=pltpu.CompilerParams(
            dimension_semantics=("arbitrary",),
        ),
    )(weightT)


def _sc_gather(w2, bias, nlp, target, samples, log_ns):
    D2 = w2.shape[1]
    B = target.shape[0]
    S = samples.shape[0]
    sh_blk = _PACK_NC.bit_length() - 1
    sh_q = _PACK_Q.bit_length() - 1
    mesh = plsc.VectorSubcoreMesh(core_axis_name="c", subcore_axis_name="s")

    @functools.partial(
        pl.kernel,
        mesh=mesh,
        out_type=[
            jax.ShapeDtypeStruct((B, D2), jnp.float32),
            jax.ShapeDtypeStruct((S, D2), jnp.float32),
            jax.ShapeDtypeStruct((B,), jnp.float32),
            jax.ShapeDtypeStruct((S,), jnp.float32),
            jax.ShapeDtypeStruct((B,), jnp.float32),
            jax.ShapeDtypeStruct((S,), jnp.float32),
        ],
        scratch_types=[
            pltpu.VMEM((_CHUNK,), jnp.int32),
            pltpu.VMEM((_CHUNK,), jnp.int32),
            pltpu.VMEM((_CHUNK, D2), jnp.float32),
            pltpu.VMEM((_CHUNK,), jnp.float32),
            pltpu.VMEM((_CHUNK,), jnp.float32),
            pltpu.VMEM((_CHUNK,), jnp.float32),
            pltpu.VMEM((_CHUNK,), jnp.float32),
            pltpu.SemaphoreType.DMA,
        ],
        compiler_params=pltpu.CompilerParams(use_tc_tiling_on_sc=False),
    )
    def gather_kernel(w2_hbm, bias_hbm, nlp_hbm, target_hbm, samples_hbm,
                      trows_hbm, srows_hbm, toff_hbm, soff_hbm,
                      tpar_hbm, spar_hbm,
                      idx_v, slot_v, rows_v, b_v, n_v, o_v, p_v, sem):
        wid = lax.axis_index("s") * _NC + lax.axis_index("c")

        def do_chunk(idx_hbm, rows_out, off_out, par_out, base):
            pltpu.sync_copy(idx_hbm.at[pl.ds(base, _CHUNK)], idx_v)
            for k in range(_CHUNK // 16):
                sl = pl.ds(k * 16, 16)
                c = idx_v[sl]
                slot_v[sl] = lax.bitwise_or(
                    lax.shift_left(lax.shift_right_logical(c, sh_blk),
                                   sh_q),
                    lax.bitwise_and(c, _PACK_Q - 1))
                p_v[sl] = lax.convert_element_type(
                    lax.bitwise_and(lax.shift_right_logical(c, sh_q), 3),
                    jnp.float32)
            pltpu.async_copy(w2_hbm.at[slot_v], rows_v, sem).wait()
            pltpu.async_copy(bias_hbm.at[idx_v], b_v, sem).wait()
            pltpu.async_copy(nlp_hbm.at[idx_v], n_v, sem).wait()
            for k in range(_CHUNK // 16):
                sl = pl.ds(k * 16, 16)
                o_v[sl] = b_v[sl] - (n_v[sl] + log_ns)
            pltpu.sync_copy(rows_v, rows_out.at[pl.ds(base, _CHUNK)])
            pltpu.sync_copy(o_v, off_out.at[pl.ds(base, _CHUNK)])
            pltpu.sync_copy(p_v, par_out.at[pl.ds(base, _CHUNK)])

        for c in range(B // (_NW * _CHUNK)):
            do_chunk(target_hbm, trows_hbm, toff_hbm, tpar_hbm,
                     wid * (B // _NW) + c * _CHUNK)
        for c in range(S // (_NW * _CHUNK)):
            do_chunk(samples_hbm, srows_hbm, soff_hbm, spar_hbm,
                     wid * (S // _NW) + c * _CHUNK)

    return gather_kernel(w2, bias, nlp, target, samples)


_LOG2E = 1.4426950408889634
_LN2 = 0.6931471805599453


def _softplus_neg_abs(x):
    p = jnp.exp2(jnp.abs(x) * (-_LOG2E))
    return _LN2 * jnp.log2(1.0 + p)


def _tc_loss_T_body(se_ref, dt_ref, tt_ref, to_ref, tp_ref, so_ref, sp_ref,
                    out_ref):
    j = pl.program_id(0)
    D = dt_ref.shape[0]
    Dh = D // 2
    dt = dt_ref[...]
    se = se_ref[...]
    q = sp_ref[...].T
    hi_l, lo_l = _unpack_bf16(se[:, :D])
    hi_u, lo_u = _unpack_bf16(se[:, D:])
    emb = jnp.where(q < 1.5,
                    jnp.where(q < 0.5, hi_l, lo_l),
                    jnp.where(q < 2.5, hi_u, lo_u))
    mm = lax.dot_general(emb, dt,
                         dimension_numbers=(((1,), (0,)), ((), ())),
                         preferred_element_type=jnp.float32)
    sl = mm + so_ref[...].T
    out_ref[...] = jnp.maximum(sl, 0.0) + _softplus_neg_abs(sl)

    @pl.when(j == 0)
    def _():
        tt = tt_ref[...]
        tq = tp_ref[...]
        thi_l, tlo_l = _unpack_bf16(tt[:D, :])
        thi_u, tlo_u = _unpack_bf16(tt[D:, :])
        temb = jnp.where(tq < 1.5,
                         jnp.where(tq < 0.5, thi_l, tlo_l),
                         jnp.where(tq < 2.5, thi_u, tlo_u))
        tl = jnp.sum(dt * temb, axis=0, keepdims=True)
        tl = tl + to_ref[...]
        out_ref[0:1, :] = jnp.maximum(tl, 0.0) - tl + _softplus_neg_abs(tl)


def _tc_loss_T(dataT, srows_p, soff_p, spar_p, trowsT, toff_row, tpar_row,
               bt=512):
    D, B = dataT.shape
    D2 = srows_p.shape[1]
    S1 = srows_p.shape[0]
    return pl.pallas_call(
        _tc_loss_T_body,
        grid=(pl.cdiv(S1, bt),),
        in_specs=[
            pl.BlockSpec((bt, D2), lambda j: (j, 0)),
            pl.BlockSpec((D, B), lambda j: (0, 0)),
            pl.BlockSpec((D2, B), lambda j: (0, 0)),
            pl.BlockSpec((1, B), lambda j: (0, 0)),
            pl.BlockSpec((1, B), lambda j: (0, 0)),
            pl.BlockSpec((1, bt), lambda j: (0, j)),
            pl.BlockSpec((1, bt), lambda j: (0, j)),
        ],
        out_specs=pl.BlockSpec((bt, B), lambda j: (j, 0)),
        out_shape=jax.ShapeDtypeStruct((S1, B), jnp.float32),
        compiler_params=pltpu.CompilerParams(
            dimension_semantics=("arbitrary",),
        ),
    )(srows_p, dataT, trowsT, toff_row, tpar_row, soff_p, spar_p)


def kernel(data, target, samples, weight, bias, noise_log_probs):
    B = data.shape[0]
    S = samples.shape[0]
    log_ns = math.log(S)
    w2 = _tc_pack(weight.T)
    trows, srows, toff, soff, tpar, spar = _sc_gather(
        w2, bias, noise_log_probs,
        target.astype(jnp.int32), samples.astype(jnp.int32), log_ns)
    srows_p = jnp.pad(srows, ((1, 0), (0, 0)))
    soff_p = jnp.pad(soff, (1, 0)).reshape(1, S + 1)
    spar_p = jnp.pad(spar, (1, 0)).reshape(1, S + 1)
    out_T = _tc_loss_T(data.T, srows_p, soff_p, spar_p, trows.T,
                       toff.reshape(1, B), tpar.reshape(1, B))
    return out_T.T


# speedup vs baseline: 2.2894x; 1.0180x over previous
import functools
import math

import jax
import jax.numpy as jnp
from jax import lax
from jax.experimental import pallas as pl
from jax.experimental.pallas import tpu as pltpu
from jax.experimental.pallas import tpu_sc as plsc

_NC = 2
_NS = 16
_NW = _NC * _NS
_CHUNK = 128

_PACK_NC = 16384
_PACK_Q = _PACK_NC // 4


def _pack_rows(V):
    tail = V - (V // _PACK_NC) * _PACK_NC
    return (V // _PACK_NC) * _PACK_Q + tail


_HI_MASK = -65536


def _pack_bf16(xhi, xlo):
    rb = lambda x: lax.bitcast_convert_type(x, jnp.int32) + 32768
    hi = lax.bitwise_and(rb(xhi), _HI_MASK)
    lo = lax.shift_right_logical(rb(xlo), 16)
    return lax.bitcast_convert_type(lax.bitwise_or(hi, lo), jnp.float32)


def _unpack_bf16(pk):
    pki = lax.bitcast_convert_type(pk, jnp.int32)
    hi = lax.bitcast_convert_type(lax.bitwise_and(pki, _HI_MASK), jnp.float32)
    lo = lax.bitcast_convert_type(lax.shift_left(pki, 16), jnp.float32)
    return hi, lo


def _pack_body(wt_ref, out_ref):
    Q = _PACK_Q
    xt = wt_ref[...].T
    a = _pack_bf16(xt[:Q], xt[Q:2 * Q])
    b = _pack_bf16(xt[2 * Q:3 * Q], xt[3 * Q:])
    out_ref[...] = jnp.concatenate([a, b], axis=1)


def _tc_pack(weightT):
    D, V = weightT.shape
    return pl.pallas_call(
        _pack_body,
        grid=(pl.cdiv(V, _PACK_NC),),
        in_specs=[pl.BlockSpec((D, _PACK_NC), lambda j: (0, j))],
        out_specs=pl.BlockSpec((_PACK_Q, 2 * D), lambda j: (j, 0)),
        out_shape=jax.ShapeDtypeStruct((_pack_rows(V), 2 * D), jnp.float32),
        compiler_params=pltpu.CompilerParams(
            dimension_semantics=("arbitrary",),
        ),
    )(weightT)


def _sc_offsets(bias, nlp, target, samples, log_ns):
    B = target.shape[0]
    S = samples.shape[0]
    sh_q = _PACK_Q.bit_length() - 1
    mesh = plsc.VectorSubcoreMesh(core_axis_name="c", subcore_axis_name="s")

    @functools.partial(
        pl.kernel,
        mesh=mesh,
        out_type=[
            jax.ShapeDtypeStruct((B,), jnp.float32),
            jax.ShapeDtypeStruct((S,), jnp.float32),
            jax.ShapeDtypeStruct((B,), jnp.float32),
            jax.ShapeDtypeStruct((S,), jnp.float32),
        ],
        scratch_types=[
            pltpu.VMEM((_CHUNK,), jnp.int32),
            pltpu.VMEM((_CHUNK,), jnp.float32),
            pltpu.VMEM((_CHUNK,), jnp.float32),
            pltpu.VMEM((_CHUNK,), jnp.float32),
            pltpu.VMEM((_CHUNK,), jnp.float32),
            pltpu.SemaphoreType.DMA,
        ],
        compiler_params=pltpu.CompilerParams(use_tc_tiling_on_sc=False),
    )
    def offsets_kernel(bias_hbm, nlp_hbm, target_hbm, samples_hbm,
                       toff_hbm, soff_hbm, tpar_hbm, spar_hbm,
                       idx_v, b_v, n_v, o_v, p_v, sem):
        wid = lax.axis_index("s") * _NC + lax.axis_index("c")

        def do_chunk(idx_hbm, off_out, par_out, base):
            pltpu.sync_copy(idx_hbm.at[pl.ds(base, _CHUNK)], idx_v)
            pltpu.async_copy(bias_hbm.at[idx_v], b_v, sem).wait()
            pltpu.async_copy(nlp_hbm.at[idx_v], n_v, sem).wait()
            for k in range(_CHUNK // 16):
                sl = pl.ds(k * 16, 16)
                c = idx_v[sl]
                o_v[sl] = b_v[sl] - (n_v[sl] + log_ns)
                p_v[sl] = lax.convert_element_type(
                    lax.bitwise_and(lax.shift_right_logical(c, sh_q), 3),
                    jnp.float32)
            pltpu.sync_copy(o_v, off_out.at[pl.ds(base, _CHUNK)])
            pltpu.sync_copy(p_v, par_out.at[pl.ds(base, _CHUNK)])

        for c in range(B // (_NW * _CHUNK)):
            do_chunk(target_hbm, toff_hbm, tpar_hbm,
                     wid * (B // _NW) + c * _CHUNK)
        for c in range(S // (_NW * _CHUNK)):
            do_chunk(samples_hbm, soff_hbm, spar_hbm,
                     wid * (S // _NW) + c * _CHUNK)

    return offsets_kernel(bias, nlp, target, samples)


def _sc_gather(w2, target, samples):
    D2 = w2.shape[1]
    B = target.shape[0]
    S = samples.shape[0]
    sh_blk = _PACK_NC.bit_length() - 1
    sh_q = _PACK_Q.bit_length() - 1
    mesh = plsc.VectorSubcoreMesh(core_axis_name="c", subcore_axis_name="s")

    @functools.partial(
        pl.kernel,
        mesh=mesh,
        out_type=[
            jax.ShapeDtypeStruct((B, D2), jnp.float32),
            jax.ShapeDtypeStruct((S, D2), jnp.float32),
        ],
        scratch_types=[
            pltpu.VMEM((_CHUNK,), jnp.int32),
            pltpu.VMEM((_CHUNK,), jnp.int32),
            pltpu.VMEM((_CHUNK, D2), jnp.float32),
            pltpu.SemaphoreType.DMA,
        ],
        compiler_params=pltpu.CompilerParams(use_tc_tiling_on_sc=False),
    )
    def gather_kernel(w2_hbm, target_hbm, samples_hbm,
                      trows_hbm, srows_hbm,
                      idx_v, slot_v, rows_v, sem):
        wid = lax.axis_index("s") * _NC + lax.axis_index("c")

        def do_chunk(idx_hbm, rows_out, base):
            pltpu.sync_copy(idx_hbm.at[pl.ds(base, _CHUNK)], idx_v)
            for k in range(_CHUNK // 16):
                sl = pl.ds(k * 16, 16)
                c = idx_v[sl]
                slot_v[sl] = lax.bitwise_or(
                    lax.shift_left(lax.shift_right_logical(c, sh_blk),
                                   sh_q),
                    lax.bitwise_and(c, _PACK_Q - 1))
            pltpu.async_copy(w2_hbm.at[slot_v], rows_v, sem).wait()
            pltpu.sync_copy(rows_v, rows_out.at[pl.ds(base, _CHUNK)])

        for c in range(B // (_NW * _CHUNK)):
            do_chunk(target_hbm, trows_hbm, wid * (B // _NW) + c * _CHUNK)
        for c in range(S // (_NW * _CHUNK)):
            do_chunk(samples_hbm, srows_hbm, wid * (S // _NW) + c * _CHUNK)

    return gather_kernel(w2, target, samples)


_LOG2E = 1.4426950408889634
_LN2 = 0.6931471805599453


def _softplus_neg_abs(x):
    p = jnp.exp2(jnp.abs(x) * (-_LOG2E))
    return _LN2 * jnp.log2(1.0 + p)


def _tc_loss_T_body(se_ref, dt_ref, tt_ref, to_ref, tp_ref, so_ref, sp_ref,
                    out_ref):
    j = pl.program_id(0)
    D = dt_ref.shape[0]
    Dh = D // 2
    dt = dt_ref[...]
    se = se_ref[...]
    q = sp_ref[...].T
    hi_l, lo_l = _unpack_bf16(se[:, :D])
    hi_u, lo_u = _unpack_bf16(se[:, D:])
    emb = jnp.where(q < 1.5,
                    jnp.where(q < 0.5, hi_l, lo_l),
                    jnp.where(q < 2.5, hi_u, lo_u))
    mm = lax.dot_general(emb, dt,
                         dimension_numbers=(((1,), (0,)), ((), ())),
                         preferred_element_type=jnp.float32)
    sl = mm + so_ref[...].T
    out_ref[...] = jnp.maximum(sl, 0.0) + _softplus_neg_abs(sl)

    @pl.when(j == 0)
    def _():
        tt = tt_ref[...]
        tq = tp_ref[...]
        thi_l, tlo_l = _unpack_bf16(tt[:D, :])
        thi_u, tlo_u = _unpack_bf16(tt[D:, :])
        temb = jnp.where(tq < 1.5,
                         jnp.where(tq < 0.5, thi_l, tlo_l),
                         jnp.where(tq < 2.5, thi_u, tlo_u))
        tl = jnp.sum(dt * temb, axis=0, keepdims=True)
        tl = tl + to_ref[...]
        out_ref[0:1, :] = jnp.maximum(tl, 0.0) - tl + _softplus_neg_abs(tl)


def _tc_loss_T(dataT, srows_p, soff_p, spar_p, trowsT, toff_row, tpar_row,
               bt=512):
    D, B = dataT.shape
    D2 = srows_p.shape[1]
    S1 = srows_p.shape[0]
    return pl.pallas_call(
        _tc_loss_T_body,
        grid=(pl.cdiv(S1, bt),),
        in_specs=[
            pl.BlockSpec((bt, D2), lambda j: (j, 0)),
            pl.BlockSpec((D, B), lambda j: (0, 0)),
            pl.BlockSpec((D2, B), lambda j: (0, 0)),
            pl.BlockSpec((1, B), lambda j: (0, 0)),
            pl.BlockSpec((1, B), lambda j: (0, 0)),
            pl.BlockSpec((1, bt), lambda j: (0, j)),
            pl.BlockSpec((1, bt), lambda j: (0, j)),
        ],
        out_specs=pl.BlockSpec((bt, B), lambda j: (j, 0)),
        out_shape=jax.ShapeDtypeStruct((S1, B), jnp.float32),
        compiler_params=pltpu.CompilerParams(
            dimension_semantics=("arbitrary",),
        ),
    )(srows_p, dataT, trowsT, toff_row, tpar_row, soff_p, spar_p)


def kernel(data, target, samples, weight, bias, noise_log_probs):
    B = data.shape[0]
    S = samples.shape[0]
    log_ns = math.log(S)
    ti = target.astype(jnp.int32)
    si = samples.astype(jnp.int32)
    toff, soff, tpar, spar = _sc_offsets(bias, noise_log_probs, ti, si,
                                         log_ns)
    w2 = _tc_pack(weight.T)
    trows, srows = _sc_gather(w2, ti, si)
    srows_p = jnp.pad(srows, ((1, 0), (0, 0)))
    soff_p = jnp.pad(soff, (1, 0)).reshape(1, S + 1)
    spar_p = jnp.pad(spar, (1, 0)).reshape(1, S + 1)
    out_T = _tc_loss_T(data.T, srows_p, soff_p, spar_p, trows.T,
                       toff.reshape(1, B), tpar.reshape(1, B))
    return out_T.T


# speedup vs baseline: 2.3000x; 1.0046x over previous
import functools
import math

import jax
import jax.numpy as jnp
from jax import lax
from jax.experimental import pallas as pl
from jax.experimental.pallas import tpu as pltpu
from jax.experimental.pallas import tpu_sc as plsc

_NC = 2
_NS = 16
_NW = _NC * _NS
_CHUNK = 128

_PACK_NC = 32768
_PACK_Q = _PACK_NC // 4


def _pack_rows(V):
    tail = V - (V // _PACK_NC) * _PACK_NC
    return (V // _PACK_NC) * _PACK_Q + tail


_HI_MASK = -65536


def _pack_bf16(xhi, xlo):
    rb = lambda x: lax.bitcast_convert_type(x, jnp.int32) + 32768
    hi = lax.bitwise_and(rb(xhi), _HI_MASK)
    lo = lax.shift_right_logical(rb(xlo), 16)
    return lax.bitcast_convert_type(lax.bitwise_or(hi, lo), jnp.float32)


def _unpack_bf16(pk):
    pki = lax.bitcast_convert_type(pk, jnp.int32)
    hi = lax.bitcast_convert_type(lax.bitwise_and(pki, _HI_MASK), jnp.float32)
    lo = lax.bitcast_convert_type(lax.shift_left(pki, 16), jnp.float32)
    return hi, lo


def _pack_body(wt_ref, out_ref):
    Q = _PACK_Q
    xt = wt_ref[...].T
    a = _pack_bf16(xt[:Q], xt[Q:2 * Q])
    b = _pack_bf16(xt[2 * Q:3 * Q], xt[3 * Q:])
    out_ref[...] = jnp.concatenate([a, b], axis=1)


def _tc_pack(weightT):
    D, V = weightT.shape
    return pl.pallas_call(
        _pack_body,
        grid=(pl.cdiv(V, _PACK_NC),),
        in_specs=[pl.BlockSpec((D, _PACK_NC), lambda j: (0, j))],
        out_specs=pl.BlockSpec((_PACK_Q, 2 * D), lambda j: (j, 0)),
        out_shape=jax.ShapeDtypeStruct((_pack_rows(V), 2 * D), jnp.float32),
        compiler_params=pltpu.CompilerParams(
            dimension_semantics=("arbitrary",),
        ),
    )(weightT)


def _sc_offsets(bias, nlp, target, samples, log_ns):
    B = target.shape[0]
    S = samples.shape[0]
    sh_q = _PACK_Q.bit_length() - 1
    mesh = plsc.VectorSubcoreMesh(core_axis_name="c", subcore_axis_name="s")

    @functools.partial(
        pl.kernel,
        mesh=mesh,
        out_type=[
            jax.ShapeDtypeStruct((B,), jnp.float32),
            jax.ShapeDtypeStruct((S,), jnp.float32),
            jax.ShapeDtypeStruct((B,), jnp.float32),
            jax.ShapeDtypeStruct((S,), jnp.float32),
        ],
        scratch_types=[
            pltpu.VMEM((_CHUNK,), jnp.int32),
            pltpu.VMEM((_CHUNK,), jnp.float32),
            pltpu.VMEM((_CHUNK,), jnp.float32),
            pltpu.VMEM((_CHUNK,), jnp.float32),
            pltpu.VMEM((_CHUNK,), jnp.float32),
            pltpu.SemaphoreType.DMA,
        ],
        compiler_params=pltpu.CompilerParams(use_tc_tiling_on_sc=False),
    )
    def offsets_kernel(bias_hbm, nlp_hbm, target_hbm, samples_hbm,
                       toff_hbm, soff_hbm, tpar_hbm, spar_hbm,
                       idx_v, b_v, n_v, o_v, p_v, sem):
        wid = lax.axis_index("s") * _NC + lax.axis_index("c")

        def do_chunk(idx_hbm, off_out, par_out, base):
            pltpu.sync_copy(idx_hbm.at[pl.ds(base, _CHUNK)], idx_v)
            pltpu.async_copy(bias_hbm.at[idx_v], b_v, sem).wait()
            pltpu.async_copy(nlp_hbm.at[idx_v], n_v, sem).wait()
            for k in range(_CHUNK // 16):
                sl = pl.ds(k * 16, 16)
                c = idx_v[sl]
                o_v[sl] = b_v[sl] - (n_v[sl] + log_ns)
                p_v[sl] = lax.convert_element_type(
                    lax.bitwise_and(lax.shift_right_logical(c, sh_q), 3),
                    jnp.float32)
            pltpu.sync_copy(o_v, off_out.at[pl.ds(base, _CHUNK)])
            pltpu.sync_copy(p_v, par_out.at[pl.ds(base, _CHUNK)])

        for c in range(B // (_NW * _CHUNK)):
            do_chunk(target_hbm, toff_hbm, tpar_hbm,
                     wid * (B // _NW) + c * _CHUNK)
        for c in range(S // (_NW * _CHUNK)):
            do_chunk(samples_hbm, soff_hbm, spar_hbm,
                     wid * (S // _NW) + c * _CHUNK)

    return offsets_kernel(bias, nlp, target, samples)


def _sc_gather(w2, target, samples):
    D2 = w2.shape[1]
    B = target.shape[0]
    S = samples.shape[0]
    sh_blk = _PACK_NC.bit_length() - 1
    sh_q = _PACK_Q.bit_length() - 1
    mesh = plsc.VectorSubcoreMesh(core_axis_name="c", subcore_axis_name="s")

    @functools.partial(
        pl.kernel,
        mesh=mesh,
        out_type=[
            jax.ShapeDtypeStruct((B, D2), jnp.float32),
            jax.ShapeDtypeStruct((S, D2), jnp.float32),
        ],
        scratch_types=[
            pltpu.VMEM((_CHUNK,), jnp.int32),
            pltpu.VMEM((_CHUNK,), jnp.int32),
            pltpu.VMEM((_CHUNK, D2), jnp.float32),
            pltpu.SemaphoreType.DMA,
        ],
        compiler_params=pltpu.CompilerParams(use_tc_tiling_on_sc=False),
    )
    def gather_kernel(w2_hbm, target_hbm, samples_hbm,
                      trows_hbm, srows_hbm,
                      idx_v, slot_v, rows_v, sem):
        wid = lax.axis_index("s") * _NC + lax.axis_index("c")

        def do_chunk(idx_hbm, rows_out, base):
            pltpu.sync_copy(idx_hbm.at[pl.ds(base, _CHUNK)], idx_v)
            for k in range(_CHUNK // 16):
                sl = pl.ds(k * 16, 16)
                c = idx_v[sl]
                slot_v[sl] = lax.bitwise_or(
                    lax.shift_left(lax.shift_right_logical(c, sh_blk),
                                   sh_q),
                    lax.bitwise_and(c, _PACK_Q - 1))
            pltpu.async_copy(w2_hbm.at[slot_v], rows_v, sem).wait()
            pltpu.sync_copy(rows_v, rows_out.at[pl.ds(base, _CHUNK)])

        for c in range(B // (_NW * _CHUNK)):
            do_chunk(target_hbm, trows_hbm, wid * (B // _NW) + c * _CHUNK)
        for c in range(S // (_NW * _CHUNK)):
            do_chunk(samples_hbm, srows_hbm, wid * (S // _NW) + c * _CHUNK)

    return gather_kernel(w2, target, samples)


_LOG2E = 1.4426950408889634
_LN2 = 0.6931471805599453


def _softplus_neg_abs(x):
    p = jnp.exp2(jnp.abs(x) * (-_LOG2E))
    return _LN2 * jnp.log2(1.0 + p)


def _tc_loss_T_body(se_ref, dt_ref, tt_ref, to_ref, tp_ref, so_ref, sp_ref,
                    out_ref):
    j = pl.program_id(0)
    D = dt_ref.shape[0]
    Dh = D // 2
    dt = dt_ref[...]
    se = se_ref[...]
    q = sp_ref[...].T
    hi_l, lo_l = _unpack_bf16(se[:, :D])
    hi_u, lo_u = _unpack_bf16(se[:, D:])
    emb = jnp.where(q < 1.5,
                    jnp.where(q < 0.5, hi_l, lo_l),
                    jnp.where(q < 2.5, hi_u, lo_u))
    mm = lax.dot_general(emb, dt,
                         dimension_numbers=(((1,), (0,)), ((), ())),
                         preferred_element_type=jnp.float32)
    sl = mm + so_ref[...].T
    out_ref[...] = jnp.maximum(sl, 0.0) + _softplus_neg_abs(sl)

    @pl.when(j == 0)
    def _():
        tt = tt_ref[...]
        tq = tp_ref[...]
        thi_l, tlo_l = _unpack_bf16(tt[:D, :])
        thi_u, tlo_u = _unpack_bf16(tt[D:, :])
        temb = jnp.where(tq < 1.5,
                         jnp.where(tq < 0.5, thi_l, tlo_l),
                         jnp.where(tq < 2.5, thi_u, tlo_u))
        tl = jnp.sum(dt * temb, axis=0, keepdims=True)
        tl = tl + to_ref[...]
        out_ref[0:1, :] = jnp.maximum(tl, 0.0) - tl + _softplus_neg_abs(tl)


def _tc_loss_T(dataT, srows_p, soff_p, spar_p, trowsT, toff_row, tpar_row,
               bt=512):
    D, B = dataT.shape
    D2 = srows_p.shape[1]
    S1 = srows_p.shape[0]
    return pl.pallas_call(
        _tc_loss_T_body,
        grid=(pl.cdiv(S1, bt),),
        in_specs=[
            pl.BlockSpec((bt, D2), lambda j: (j, 0)),
            pl.BlockSpec((D, B), lambda j: (0, 0)),
            pl.BlockSpec((D2, B), lambda j: (0, 0)),
            pl.BlockSpec((1, B), lambda j: (0, 0)),
            pl.BlockSpec((1, B), lambda j: (0, 0)),
            pl.BlockSpec((1, bt), lambda j: (0, j)),
            pl.BlockSpec((1, bt), lambda j: (0, j)),
        ],
        out_specs=pl.BlockSpec((bt, B), lambda j: (j, 0)),
        out_shape=jax.ShapeDtypeStruct((S1, B), jnp.float32),
        compiler_params=pltpu.CompilerParams(
            dimension_semantics=("arbitrary",),
        ),
    )(srows_p, dataT, trowsT, toff_row, tpar_row, soff_p, spar_p)


def kernel(data, target, samples, weight, bias, noise_log_probs):
    B = data.shape[0]
    S = samples.shape[0]
    log_ns = math.log(S)
    ti = target.astype(jnp.int32)
    si = samples.astype(jnp.int32)
    toff, soff, tpar, spar = _sc_offsets(bias, noise_log_probs, ti, si,
                                         log_ns)
    w2 = _tc_pack(weight.T)
    trows, srows = _sc_gather(w2, ti, si)
    srows_p = jnp.pad(srows, ((1, 0), (0, 0)))
    soff_p = jnp.pad(soff, (1, 0)).reshape(1, S + 1)
    spar_p = jnp.pad(spar, (1, 0)).reshape(1, S + 1)
    out_T = _tc_loss_T(data.T, srows_p, soff_p, spar_p, trows.T,
                       toff.reshape(1, B), tpar.reshape(1, B))
    return out_T.T


# speedup vs baseline: 2.3039x; 1.0017x over previous
import functools
import math

import jax
import jax.numpy as jnp
from jax import lax
from jax.experimental import pallas as pl
from jax.experimental.pallas import tpu as pltpu
from jax.experimental.pallas import tpu_sc as plsc

_NC = 2
_NS = 16
_NW = _NC * _NS
_CHUNK = 128

_PACK_NC = 32768
_PACK_Q = _PACK_NC // 4


def _pack_rows(V):
    tail = V - (V // _PACK_NC) * _PACK_NC
    return (V // _PACK_NC) * _PACK_Q + tail


_HI_MASK = -65536


def _pack_bf16(xhi, xlo):
    rb = lambda x: lax.bitcast_convert_type(x, jnp.int32) + 32768
    hi = lax.bitwise_and(rb(xhi), _HI_MASK)
    lo = lax.shift_right_logical(rb(xlo), 16)
    return lax.bitcast_convert_type(lax.bitwise_or(hi, lo), jnp.float32)


def _unpack_bf16(pk):
    pki = lax.bitcast_convert_type(pk, jnp.int32)
    hi = lax.bitcast_convert_type(lax.bitwise_and(pki, _HI_MASK), jnp.float32)
    lo = lax.bitcast_convert_type(lax.shift_left(pki, 16), jnp.float32)
    return hi, lo


def _pack_body(wt_ref, out_ref):
    Q = _PACK_Q
    xt = wt_ref[...].T
    a = _pack_bf16(xt[:Q], xt[Q:2 * Q])
    b = _pack_bf16(xt[2 * Q:3 * Q], xt[3 * Q:])
    out_ref[...] = jnp.concatenate([a, b], axis=1)


def _tc_pack(weightT):
    D, V = weightT.shape
    return pl.pallas_call(
        _pack_body,
        grid=(pl.cdiv(V, _PACK_NC),),
        in_specs=[pl.BlockSpec((D, _PACK_NC), lambda j: (0, j))],
        out_specs=pl.BlockSpec((_PACK_Q, 2 * D), lambda j: (j, 0)),
        out_shape=jax.ShapeDtypeStruct((_pack_rows(V), 2 * D), jnp.float32),
        compiler_params=pltpu.CompilerParams(
            dimension_semantics=("arbitrary",),
        ),
    )(weightT)


def _sc_offsets(bias, nlp, target, samples, log_ns):
    B = target.shape[0]
    S = samples.shape[0]
    sh_q = _PACK_Q.bit_length() - 1
    mesh = plsc.VectorSubcoreMesh(core_axis_name="c", subcore_axis_name="s")

    @functools.partial(
        pl.kernel,
        mesh=mesh,
        out_type=[
            jax.ShapeDtypeStruct((B,), jnp.float32),
            jax.ShapeDtypeStruct((S,), jnp.float32),
            jax.ShapeDtypeStruct((B,), jnp.float32),
            jax.ShapeDtypeStruct((S,), jnp.float32),
        ],
        scratch_types=[
            pltpu.VMEM((_CHUNK,), jnp.int32),
            pltpu.VMEM((_CHUNK,), jnp.float32),
            pltpu.VMEM((_CHUNK,), jnp.float32),
            pltpu.VMEM((_CHUNK,), jnp.float32),
            pltpu.VMEM((_CHUNK,), jnp.float32),
            pltpu.SemaphoreType.DMA,
        ],
        compiler_params=pltpu.CompilerParams(use_tc_tiling_on_sc=False),
    )
    def offsets_kernel(bias_hbm, nlp_hbm, target_hbm, samples_hbm,
                       toff_hbm, soff_hbm, tpar_hbm, spar_hbm,
                       idx_v, b_v, n_v, o_v, p_v, sem):
        wid = lax.axis_index("s") * _NC + lax.axis_index("c")

        def do_chunk(idx_hbm, off_out, par_out, base):
            pltpu.sync_copy(idx_hbm.at[pl.ds(base, _CHUNK)], idx_v)
            pltpu.async_copy(bias_hbm.at[idx_v], b_v, sem).wait()
            pltpu.async_copy(nlp_hbm.at[idx_v], n_v, sem).wait()
            for k in range(_CHUNK // 16):
                sl = pl.ds(k * 16, 16)
                c = idx_v[sl]
                o_v[sl] = b_v[sl] - (n_v[sl] + log_ns)
                p_v[sl] = lax.convert_element_type(
                    lax.bitwise_and(lax.shift_right_logical(c, sh_q), 3),
                    jnp.float32)
            pltpu.sync_copy(o_v, off_out.at[pl.ds(base, _CHUNK)])
            pltpu.sync_copy(p_v, par_out.at[pl.ds(base, _CHUNK)])

        for c in range(B // (_NW * _CHUNK)):
            do_chunk(target_hbm, toff_hbm, tpar_hbm,
                     wid * (B // _NW) + c * _CHUNK)
        for c in range(S // (_NW * _CHUNK)):
            do_chunk(samples_hbm, soff_hbm, spar_hbm,
                     wid * (S // _NW) + c * _CHUNK)

    return offsets_kernel(bias, nlp, target, samples)


def _sc_gather(w2, target, samples):
    D2 = w2.shape[1]
    B = target.shape[0]
    S = samples.shape[0]
    sh_blk = _PACK_NC.bit_length() - 1
    sh_q = _PACK_Q.bit_length() - 1
    mesh = plsc.VectorSubcoreMesh(core_axis_name="c", subcore_axis_name="s")

    @functools.partial(
        pl.kernel,
        mesh=mesh,
        out_type=[
            jax.ShapeDtypeStruct((B, D2), jnp.float32),
            jax.ShapeDtypeStruct((S, D2), jnp.float32),
        ],
        scratch_types=[
            pltpu.VMEM((_CHUNK,), jnp.int32),
            pltpu.VMEM((_CHUNK,), jnp.int32),
            pltpu.VMEM((_CHUNK, D2), jnp.float32),
            pltpu.SemaphoreType.DMA,
        ],
        compiler_params=pltpu.CompilerParams(use_tc_tiling_on_sc=False),
    )
    def gather_kernel(w2_hbm, target_hbm, samples_hbm,
                      trows_hbm, srows_hbm,
                      idx_v, slot_v, rows_v, sem):
        wid = lax.axis_index("s") * _NC + lax.axis_index("c")

        def do_chunk(idx_hbm, rows_out, base):
            pltpu.sync_copy(idx_hbm.at[pl.ds(base, _CHUNK)], idx_v)
            for k in range(_CHUNK // 16):
                sl = pl.ds(k * 16, 16)
                c = idx_v[sl]
                slot_v[sl] = lax.bitwise_or(
                    lax.shift_left(lax.shift_right_logical(c, sh_blk),
                                   sh_q),
                    lax.bitwise_and(c, _PACK_Q - 1))
            pltpu.async_copy(w2_hbm.at[slot_v], rows_v, sem).wait()
            pltpu.sync_copy(rows_v, rows_out.at[pl.ds(base, _CHUNK)])

        for c in range(B // (_NW * _CHUNK)):
            do_chunk(target_hbm, trows_hbm, wid * (B // _NW) + c * _CHUNK)
        for c in range(S // (_NW * _CHUNK)):
            do_chunk(samples_hbm, srows_hbm, wid * (S // _NW) + c * _CHUNK)

    return gather_kernel(w2, target, samples)


_LOG2E = 1.4426950408889634
_LN2 = 0.6931471805599453


def _softplus_neg_abs(x):
    p = jnp.exp2(jnp.abs(x) * (-_LOG2E))
    return _LN2 * jnp.log2(1.0 + p)


def _tc_loss_T_body(se_ref, dt_ref, tt_ref, to_ref, tp_ref, so_ref, sp_ref,
                    out_ref):
    j = pl.program_id(0)
    D = dt_ref.shape[0]
    dt = dt_ref[...]
    se = se_ref[...]
    q = sp_ref[...].T
    hi_l, lo_l = _unpack_bf16(se[:, :D])
    hi_u, lo_u = _unpack_bf16(se[:, D:])
    emb = jnp.where(q < 1.5,
                    jnp.where(q < 0.5, hi_l, lo_l),
                    jnp.where(q < 2.5, hi_u, lo_u))
    mm = lax.dot_general(emb, dt,
                         dimension_numbers=(((1,), (0,)), ((), ())),
                         preferred_element_type=jnp.float32)
    sl = mm + so_ref[...].T
    out_ref[...] = jnp.maximum(sl, 0.0) + _softplus_neg_abs(sl)

    @pl.when(j == 0)
    def _():
        tt = tt_ref[...]
        tq = tp_ref[...]
        thi_l, tlo_l = _unpack_bf16(tt[:D, :])
        thi_u, tlo_u = _unpack_bf16(tt[D:, :])
        temb = jnp.where(tq < 1.5,
                         jnp.where(tq < 0.5, thi_l, tlo_l),
                         jnp.where(tq < 2.5, thi_u, tlo_u))
        tl = jnp.sum(dt * temb, axis=0, keepdims=True)
        tl = tl + to_ref[...]
        out_ref[0:1, :] = jnp.maximum(tl, 0.0) - tl + _softplus_neg_abs(tl)


def _tc_loss_T(dataT, srows_p, soff_p, spar_p, trowsT, toff_row, tpar_row,
               bt=512):
    D, B = dataT.shape
    D2 = srows_p.shape[1]
    S1 = srows_p.shape[0]
    return pl.pallas_call(
        _tc_loss_T_body,
        grid=(pl.cdiv(S1, bt),),
        in_specs=[
            pl.BlockSpec((bt, D2), lambda j: (j, 0)),
            pl.BlockSpec((D, B), lambda j: (0, 0)),
            pl.BlockSpec((D2, B), lambda j: (0, 0)),
            pl.BlockSpec((1, B), lambda j: (0, 0)),
            pl.BlockSpec((1, B), lambda j: (0, 0)),
            pl.BlockSpec((1, bt), lambda j: (0, j)),
            pl.BlockSpec((1, bt), lambda j: (0, j)),
        ],
        out_specs=pl.BlockSpec((bt, B), lambda j: (j, 0)),
        out_shape=jax.ShapeDtypeStruct((S1, B), jnp.float32),
        compiler_params=pltpu.CompilerParams(
            dimension_semantics=("arbitrary",),
        ),
    )(srows_p, dataT, trowsT, toff_row, tpar_row, soff_p, spar_p)


def kernel(data, target, samples, weight, bias, noise_log_probs):
    B = data.shape[0]
    S = samples.shape[0]
    log_ns = math.log(S)
    ti = target.astype(jnp.int32)
    si = samples.astype(jnp.int32)
    toff, soff, tpar, spar = _sc_offsets(bias, noise_log_probs, ti, si,
                                         log_ns)
    w2 = _tc_pack(weight.T)
    trows, srows = _sc_gather(w2, ti, si)
    srows_p = jnp.pad(srows, ((1, 0), (0, 0)))
    soff_p = jnp.pad(soff, (1, 0)).reshape(1, S + 1)
    spar_p = jnp.pad(spar, (1, 0)).reshape(1, S + 1)
    out_T = _tc_loss_T(data.T, srows_p, soff_p, spar_p, trows.T,
                       toff.reshape(1, B), tpar.reshape(1, B))
    return out_T.T
